# baseline (speedup 1.0000x reference)
import functools

import jax
import jax.numpy as jnp
from jax import lax
from jax.experimental import pallas as pl
from jax.experimental.pallas import tpu as pltpu
from jax.experimental.pallas import tpu_sc as plsc

ENT = 1000000
REL = 1000
H = 64
B = 4096
MARGIN = 1.0

NC = 2
NS = 16
NW = NC * NS
ROWS_PER_W = 4 * B // NW


def _sc_gather_body(idx_hbm, table_hbm, out_hbm, idx_v, rows_v, sem):
    wid = lax.axis_index("s") * NC + lax.axis_index("c")
    base = wid * ROWS_PER_W
    pltpu.sync_copy(idx_hbm.at[pl.ds(base, ROWS_PER_W)], idx_v)
    pltpu.async_copy(table_hbm.at[idx_v], rows_v, sem).wait()
    pltpu.sync_copy(rows_v, out_hbm.at[pl.ds(base, ROWS_PER_W)])


def _sc_gather(idx, table):
    mesh = plsc.VectorSubcoreMesh(core_axis_name="c", subcore_axis_name="s")
    return pl.kernel(
        _sc_gather_body,
        mesh=mesh,
        out_type=jax.ShapeDtypeStruct((4 * B, H), jnp.float32),
        scratch_types=[
            pltpu.VMEM((ROWS_PER_W,), jnp.int32),
            pltpu.VMEM((ROWS_PER_W, H), jnp.float32),
            pltpu.SemaphoreType.DMA,
        ],
        compiler_params=pltpu.CompilerParams(use_tc_tiling_on_sc=False),
    )(idx, table)


def _score_body(pos_r_ref, neg_r_ref, ph_ref, pt_ref, nh_ref, nt_ref,
                rel_ref, out_ref):
    def body(b, acc):
        rp = pos_r_ref[b]
        rn = neg_r_ref[b]
        Rp = rel_ref[pl.ds(rp * H, H), :]
        Rn = rel_ref[pl.ds(rn * H, H), :]
        tp = pt_ref[pl.ds(b, 1), :]
        tn = nt_ref[pl.ds(b, 1), :]
        hp = ph_ref[pl.ds(b, 1), :]
        hn = nh_ref[pl.ds(b, 1), :]
        qp = jnp.sum(Rp * tp, axis=1)
        qn = jnp.sum(Rn * tn, axis=1)
        p_score = jnp.sum(qp * hp[0, :])
        n_score = jnp.sum(qn * hn[0, :])
        return acc + jnp.maximum(n_score - p_score + MARGIN, 0.0)

    out_ref[0, 0] = lax.fori_loop(0, B, body, jnp.float32(0.0))


def _score(pos_r, neg_r, ph, pt, nh, nt, rel_v):
    return pl.pallas_call(
        _score_body,
        out_shape=jax.ShapeDtypeStruct((1, 1), jnp.float32),
        in_specs=[
            pl.BlockSpec(memory_space=pltpu.SMEM),
            pl.BlockSpec(memory_space=pltpu.SMEM),
            pl.BlockSpec(memory_space=pltpu.VMEM),
            pl.BlockSpec(memory_space=pltpu.VMEM),
            pl.BlockSpec(memory_space=pltpu.VMEM),
            pl.BlockSpec(memory_space=pltpu.VMEM),
            pl.BlockSpec(memory_space=pltpu.VMEM),
        ],
        out_specs=pl.BlockSpec(memory_space=pltpu.SMEM),
    )(pos_r, neg_r, ph, pt, nh, nt, rel_v)


def kernel(pos_h, pos_t, pos_r, neg_h, neg_t, neg_r,
           ent_embeddings, rel_matrices):
    idx = jnp.concatenate([pos_h, pos_t, neg_h, neg_t]).astype(jnp.int32)
    rows = _sc_gather(idx, ent_embeddings)
    ph = rows[0 * B:1 * B]
    pt = rows[1 * B:2 * B]
    nh = rows[2 * B:3 * B]
    nt = rows[3 * B:4 * B]
    rel_v = rel_matrices.reshape(REL * H, H)
    out = _score(pos_r.astype(jnp.int32), neg_r.astype(jnp.int32),
                 ph, pt, nh, nt, rel_v)
    return out[0, 0]


# speedup vs baseline: 2.5195x; 2.5195x over previous
import functools

import jax
import jax.numpy as jnp
from jax import lax
from jax.experimental import pallas as pl
from jax.experimental.pallas import tpu as pltpu
from jax.experimental.pallas import tpu_sc as plsc

ENT = 1000000
REL = 1000
H = 64
B = 4096
MARGIN = 1.0

NC = 2
NS = 16
NW = NC * NS
ROWS_PER_W = 4 * B // NW


def _sc_gather_body(idx_hbm, table_hbm, out_hbm, idx_v, rows_v, sem):
    wid = lax.axis_index("s") * NC + lax.axis_index("c")
    base = wid * ROWS_PER_W
    pltpu.sync_copy(idx_hbm.at[pl.ds(base, ROWS_PER_W)], idx_v)
    pltpu.async_copy(table_hbm.at[idx_v], rows_v, sem).wait()
    pltpu.sync_copy(rows_v, out_hbm.at[pl.ds(base, ROWS_PER_W)])


def _sc_gather(idx, table):
    mesh = plsc.VectorSubcoreMesh(core_axis_name="c", subcore_axis_name="s")
    return pl.kernel(
        _sc_gather_body,
        mesh=mesh,
        out_type=jax.ShapeDtypeStruct((4 * B, H), jnp.float32),
        scratch_types=[
            pltpu.VMEM((ROWS_PER_W,), jnp.int32),
            pltpu.VMEM((ROWS_PER_W, H), jnp.float32),
            pltpu.SemaphoreType.DMA,
        ],
        compiler_params=pltpu.CompilerParams(use_tc_tiling_on_sc=False),
    )(idx, table)


KU = 8


def _score_body(pos_r_ref, neg_r_ref, ph_ref, pt_ref, nh_ref, nt_ref,
                rel_ref, out_ref, diff_ref):
    def body(i, acc):
        b0 = i * KU
        for u in range(KU):
            b = b0 + u
            rp = pos_r_ref[b]
            rn = neg_r_ref[b]
            Rp = rel_ref[pl.ds(rp * H, H), :]
            Rn = rel_ref[pl.ds(rn * H, H), :]
            hp = ph_ref[pl.ds(b, 1), :]
            hn = nh_ref[pl.ds(b, 1), :]
            tp = pt_ref[pl.ds(b, 1), :]
            tn = nt_ref[pl.ds(b, 1), :]
            pvec = jnp.dot(hp, Rp, preferred_element_type=jnp.float32) * tp
            nvec = jnp.dot(hn, Rn, preferred_element_type=jnp.float32) * tn
            diff_ref[pl.ds(b, 1), :] = nvec - pvec
        return acc

    lax.fori_loop(0, B // KU, body, jnp.float32(0.0))
    d = diff_ref[...]
    s = jnp.sum(d, axis=1) + MARGIN
    out_ref[0, 0] = jnp.sum(jnp.maximum(s, 0.0))


def _score(pos_r, neg_r, ph, pt, nh, nt, rel_v):
    return pl.pallas_call(
        _score_body,
        out_shape=jax.ShapeDtypeStruct((1, 1), jnp.float32),
        in_specs=[
            pl.BlockSpec(memory_space=pltpu.SMEM),
            pl.BlockSpec(memory_space=pltpu.SMEM),
            pl.BlockSpec(memory_space=pltpu.VMEM),
            pl.BlockSpec(memory_space=pltpu.VMEM),
            pl.BlockSpec(memory_space=pltpu.VMEM),
            pl.BlockSpec(memory_space=pltpu.VMEM),
            pl.BlockSpec(memory_space=pltpu.VMEM),
        ],
        out_specs=pl.BlockSpec(memory_space=pltpu.SMEM),
        scratch_shapes=[pltpu.VMEM((B, H), jnp.float32)],
    )(pos_r, neg_r, ph, pt, nh, nt, rel_v)


def kernel(pos_h, pos_t, pos_r, neg_h, neg_t, neg_r,
           ent_embeddings, rel_matrices):
    idx = jnp.concatenate([pos_h, pos_t, neg_h, neg_t]).astype(jnp.int32)
    rows = _sc_gather(idx, ent_embeddings)
    ph = rows[0 * B:1 * B]
    pt = rows[1 * B:2 * B]
    nh = rows[2 * B:3 * B]
    nt = rows[3 * B:4 * B]
    rel_v = rel_matrices.reshape(REL * H, H)
    out = _score(pos_r.astype(jnp.int32), neg_r.astype(jnp.int32),
                 ph, pt, nh, nt, rel_v)
    return out[0, 0]
